# baseline (speedup 1.0000x reference)
import jax
import jax.numpy as jnp
from jax import lax
from jax.experimental import pallas as pl
from jax.experimental.pallas import tpu as pltpu

_D = 32
_K = 8192
_N = 8192
_COMMITMENT = 0.25
_TILE_M = 256
_CHUNK = 2048


def _vq_body(x_ref, x2_ref, cn_ref, emb_ref,
             idx_ref, qst_ref, loss_ref, perp_ref, counts_ref,
             sse_ref):
    i = pl.program_id(0)
    n = pl.num_programs(0)
    x = x_ref[...]
    emb = emb_ref[...]
    s = lax.dot_general(x, emb, (((1,), (0,)), ((), ())),
                        preferred_element_type=jnp.float32)
    dist = (x2_ref[...] - 2.0 * s) + cn_ref[...]
    neg = -dist
    iota = lax.broadcasted_iota(jnp.int32, dist.shape, 1)
    best_v = jnp.full((neg.shape[0], 1), -jnp.inf, jnp.float32)
    best_i = jnp.zeros((neg.shape[0], 1), jnp.int32)
    for t in range(_K // _CHUNK):
        sl = neg[:, t * _CHUNK:(t + 1) * _CHUNK]
        io = iota[:, t * _CHUNK:(t + 1) * _CHUNK]
        m = jnp.max(sl, axis=1, keepdims=True)
        ci = jnp.min(jnp.where(sl == m, io, _K), axis=1, keepdims=True)
        take = m > best_v
        best_v = jnp.where(take, m.astype(jnp.bfloat16).astype(jnp.float32),
                           best_v)
        best_i = jnp.where(take, ci, best_i)
    idx = best_i[:, 0]
    idx_ref[0, 0, :] = idx
    oh = (iota == idx[:, None]).astype(jnp.float32)
    q = lax.dot_general(oh, emb, (((1,), (1,)), ((), ())),
                        preferred_element_type=jnp.float32)
    qst_ref[...] = x + (q - x)
    diff = q - x
    part = jnp.sum(diff * diff)
    csum = jnp.sum(oh, axis=0, keepdims=True)

    @pl.when(i == 0)
    def _init():
        counts_ref[...] = csum
        sse_ref[0] = part

    @pl.when(i > 0)
    def _acc():
        counts_ref[...] = counts_ref[...] + csum
        sse_ref[0] = sse_ref[0] + part

    @pl.when(i == n - 1)
    def _fin():
        m = sse_ref[0] / (_N * _D)
        loss_ref[...] = jnp.broadcast_to(m + _COMMITMENT * m, (1, 1))
        p = counts_ref[...] * (1.0 / _N)
        ent = jnp.sum(p * jnp.log(p + 1e-10), axis=1, keepdims=True)
        perp_ref[...] = jnp.exp(-ent)


def kernel(x, embedding):
    xf = x.reshape(-1, _D)
    x2 = jnp.sum(xf ** 2, axis=1, keepdims=True)
    cn = jnp.sum(embedding ** 2, axis=0, keepdims=True)
    grid = xf.shape[0] // _TILE_M
    idx3, qst, loss, perp, _ = pl.pallas_call(
        _vq_body,
        grid=(grid,),
        in_specs=[
            pl.BlockSpec((_TILE_M, _D), lambda i: (i, 0)),
            pl.BlockSpec((_TILE_M, 1), lambda i: (i, 0)),
            pl.BlockSpec((1, _K), lambda i: (0, 0)),
            pl.BlockSpec((_D, _K), lambda i: (0, 0)),
        ],
        out_specs=(
            pl.BlockSpec((1, 1, _TILE_M), lambda i: (i, 0, 0)),
            pl.BlockSpec((_TILE_M, _D), lambda i: (i, 0)),
            pl.BlockSpec((1, 1), lambda i: (0, 0)),
            pl.BlockSpec((1, 1), lambda i: (0, 0)),
            pl.BlockSpec((1, _K), lambda i: (0, 0)),
        ),
        out_shape=(
            jax.ShapeDtypeStruct((grid, 1, _TILE_M), jnp.int32),
            jax.ShapeDtypeStruct((_N, _D), jnp.float32),
            jax.ShapeDtypeStruct((1, 1), jnp.float32),
            jax.ShapeDtypeStruct((1, 1), jnp.float32),
            jax.ShapeDtypeStruct((1, _K), jnp.float32),
        ),
        scratch_shapes=[pltpu.SMEM((1,), jnp.float32)],
    )(xf, x2, cn, embedding)
    return (loss[0, 0], qst.reshape(x.shape), perp[0, 0],
            idx3.reshape(x.shape[:-1]))


# speedup vs baseline: 1.0325x; 1.0325x over previous
import jax
import jax.numpy as jnp
from jax import lax
from jax.experimental import pallas as pl
from jax.experimental.pallas import tpu as pltpu

_D = 32
_K = 8192
_N = 8192
_COMMITMENT = 0.25
_TILE_M = 256
_CHUNK = 2048


def _vq_body(x_ref, x2_ref, cn_ref, emb_ref,
             idx_ref, qst_ref, loss_ref, perp_ref, counts_ref,
             sse_ref):
    i = pl.program_id(0)
    n = pl.num_programs(0)
    x = x_ref[...]
    emb = emb_ref[...]
    s_n2 = lax.dot_general((-2.0 * x), emb, (((1,), (0,)), ((), ())),
                           preferred_element_type=jnp.float32)
    dist = (x2_ref[...] + s_n2) + cn_ref[...]
    iota = lax.broadcasted_iota(jnp.int32, dist.shape, 1)
    best_v = jnp.full((dist.shape[0], 1), -jnp.inf, jnp.float32)
    best_i = jnp.zeros((dist.shape[0], 1), jnp.int32)
    for t in range(_K // _CHUNK):
        sl = dist[:, t * _CHUNK:(t + 1) * _CHUNK]
        io = iota[:, t * _CHUNK:(t + 1) * _CHUNK]
        mn = jnp.min(sl, axis=1, keepdims=True)
        ci = jnp.min(jnp.where(sl == mn, io, _K), axis=1, keepdims=True)
        m = -mn
        take = m > best_v
        best_v = jnp.where(take, m.astype(jnp.bfloat16).astype(jnp.float32),
                           best_v)
        best_i = jnp.where(take, ci, best_i)
    idx = best_i[:, 0]
    idx_ref[0, 0, :] = idx
    oh = (iota == idx[:, None]).astype(jnp.float32)
    q = lax.dot_general(oh, emb, (((1,), (1,)), ((), ())),
                        preferred_element_type=jnp.float32)
    qst_ref[...] = x + (q - x)
    diff = q - x
    part = jnp.sum(diff * diff)
    csum = jnp.sum(oh, axis=0, keepdims=True)

    @pl.when(i == 0)
    def _init():
        counts_ref[...] = csum
        sse_ref[0] = part

    @pl.when(i > 0)
    def _acc():
        counts_ref[...] = counts_ref[...] + csum
        sse_ref[0] = sse_ref[0] + part

    @pl.when(i == n - 1)
    def _fin():
        m = sse_ref[0] / (_N * _D)
        loss_ref[...] = jnp.broadcast_to(m + _COMMITMENT * m, (1, 1))
        p = counts_ref[...] * (1.0 / _N)
        ent = jnp.sum(p * jnp.log(p + 1e-10), axis=1, keepdims=True)
        perp_ref[...] = jnp.exp(-ent)


def kernel(x, embedding):
    xf = x.reshape(-1, _D)
    x2 = jnp.sum(xf ** 2, axis=1, keepdims=True)
    cn = jnp.sum(embedding ** 2, axis=0, keepdims=True)
    grid = xf.shape[0] // _TILE_M
    idx3, qst, loss, perp, _ = pl.pallas_call(
        _vq_body,
        grid=(grid,),
        in_specs=[
            pl.BlockSpec((_TILE_M, _D), lambda i: (i, 0)),
            pl.BlockSpec((_TILE_M, 1), lambda i: (i, 0)),
            pl.BlockSpec((1, _K), lambda i: (0, 0)),
            pl.BlockSpec((_D, _K), lambda i: (0, 0)),
        ],
        out_specs=(
            pl.BlockSpec((1, 1, _TILE_M), lambda i: (i, 0, 0)),
            pl.BlockSpec((_TILE_M, _D), lambda i: (i, 0)),
            pl.BlockSpec((1, 1), lambda i: (0, 0)),
            pl.BlockSpec((1, 1), lambda i: (0, 0)),
            pl.BlockSpec((1, _K), lambda i: (0, 0)),
        ),
        out_shape=(
            jax.ShapeDtypeStruct((grid, 1, _TILE_M), jnp.int32),
            jax.ShapeDtypeStruct((_N, _D), jnp.float32),
            jax.ShapeDtypeStruct((1, 1), jnp.float32),
            jax.ShapeDtypeStruct((1, 1), jnp.float32),
            jax.ShapeDtypeStruct((1, _K), jnp.float32),
        ),
        scratch_shapes=[pltpu.SMEM((1,), jnp.float32)],
    )(xf, x2, cn, embedding)
    return (loss[0, 0], qst.reshape(x.shape), perp[0, 0],
            idx3.reshape(x.shape[:-1]))


# speedup vs baseline: 1.3557x; 1.3130x over previous
import functools

import jax
import jax.numpy as jnp
from jax import lax
from jax.experimental import pallas as pl
from jax.experimental.pallas import tpu as pltpu
from jax.experimental.pallas import tpu_sc as plsc

_D = 32
_K = 8192
_N = 8192
_COMMITMENT = 0.25
_TILE_M = 256
_CHUNK = 2048
_TILE_M2 = 1024


def _argmin_body(x_ref, x2_ref, cn_ref, emb_ref, idx_ref):
    x = x_ref[...]
    emb = emb_ref[...]
    s_n2 = lax.dot_general((-2.0 * x), emb, (((1,), (0,)), ((), ())),
                           preferred_element_type=jnp.float32)
    dist = (x2_ref[...] + s_n2) + cn_ref[...]
    iota = lax.broadcasted_iota(jnp.int32, dist.shape, 1)
    best_v = jnp.full((dist.shape[0], 1), -jnp.inf, jnp.float32)
    best_i = jnp.zeros((dist.shape[0], 1), jnp.int32)
    for t in range(_K // _CHUNK):
        sl = dist[:, t * _CHUNK:(t + 1) * _CHUNK]
        io = iota[:, t * _CHUNK:(t + 1) * _CHUNK]
        mn = jnp.min(sl, axis=1, keepdims=True)
        ci = jnp.min(jnp.where(sl == mn, io, _K), axis=1, keepdims=True)
        m = -mn
        take = m > best_v
        best_v = jnp.where(take, m.astype(jnp.bfloat16).astype(jnp.float32),
                           best_v)
        best_i = jnp.where(take, ci, best_i)
    idx_ref[0, 0, :] = best_i[:, 0]


def _tc_argmin(xf, x2, cn, embedding):
    grid = _N // _TILE_M
    return pl.pallas_call(
        _argmin_body,
        grid=(grid,),
        in_specs=[
            pl.BlockSpec((_TILE_M, _D), lambda i: (i, 0)),
            pl.BlockSpec((_TILE_M, 1), lambda i: (i, 0)),
            pl.BlockSpec((1, _K), lambda i: (0, 0)),
            pl.BlockSpec((_D, _K), lambda i: (0, 0)),
        ],
        out_specs=pl.BlockSpec((1, 1, _TILE_M), lambda i: (i, 0, 0)),
        out_shape=jax.ShapeDtypeStruct((grid, 1, _TILE_M), jnp.int32),
    )(xf, x2, cn, embedding)


def _make_sc_gather_hist():
    info = plsc.get_sparse_core_info()
    nc, ns = info.num_cores, info.num_subcores
    nw = nc * ns
    bpw = _N // nw
    mesh = plsc.VectorSubcoreMesh(core_axis_name="c", subcore_axis_name="s")

    @functools.partial(
        pl.kernel, mesh=mesh,
        compiler_params=pltpu.CompilerParams(use_tc_tiling_on_sc=False),
        out_type=(jax.ShapeDtypeStruct((_N, _D), jnp.float32),
                  jax.ShapeDtypeStruct((nc, _K), jnp.float32)),
        scratch_types=[
            pltpu.VMEM((bpw,), jnp.int32),
            pltpu.VMEM((bpw, _D), jnp.float32),
            pltpu.VMEM((bpw,), jnp.float32),
            pltpu.VMEM_SHARED((_K,), jnp.float32),
            pltpu.SemaphoreType.DMA,
        ],
    )
    def sc_kernel(table_hbm, idx_hbm, ones_hbm, zeros_hbm,
                  q_hbm, counts_hbm, idx_v, rows_v, ones_v, shared, sem):
        cid = lax.axis_index("c")
        sid = lax.axis_index("s")
        wid = sid * nc + cid
        base = wid * bpw
        @pl.when(sid == 0)
        def _():
            pltpu.sync_copy(zeros_hbm, shared)
        pltpu.sync_copy(idx_hbm.at[pl.ds(base, bpw)], idx_v)
        pltpu.sync_copy(ones_hbm.at[pl.ds(base, bpw)], ones_v)
        pltpu.async_copy(table_hbm.at[idx_v], rows_v, sem).wait()
        pltpu.sync_copy(rows_v, q_hbm.at[pl.ds(base, bpw)])
        plsc.subcore_barrier()
        pltpu.sync_copy(ones_v, shared.at[idx_v], add=True)
        plsc.subcore_barrier()
        @pl.when(sid == 0)
        def _():
            pltpu.sync_copy(shared, counts_hbm.at[cid])

    return sc_kernel, nc


def _finish_body(x_ref, q_ref, counts_ref, qst_ref, loss_ref, perp_ref,
                 sse_ref):
    i = pl.program_id(0)
    n = pl.num_programs(0)
    x = x_ref[...]
    q = q_ref[...]
    qst_ref[...] = x + (q - x)
    diff = q - x
    part = jnp.sum(diff * diff)

    @pl.when(i == 0)
    def _():
        sse_ref[0] = part

    @pl.when(i > 0)
    def _():
        sse_ref[0] = sse_ref[0] + part

    @pl.when(i == n - 1)
    def _():
        m = sse_ref[0] / (_N * _D)
        loss_ref[...] = jnp.broadcast_to(m + _COMMITMENT * m, (1, 1))
        counts = counts_ref[0:1, :] + counts_ref[1:2, :]
        p = counts * (1.0 / _N)
        ent = jnp.sum(p * jnp.log(p + 1e-10), axis=1, keepdims=True)
        perp_ref[...] = jnp.exp(-ent)


def _tc_finish(xf, q, counts2):
    grid = _N // _TILE_M2
    return pl.pallas_call(
        _finish_body,
        grid=(grid,),
        in_specs=[
            pl.BlockSpec((_TILE_M2, _D), lambda i: (i, 0)),
            pl.BlockSpec((_TILE_M2, _D), lambda i: (i, 0)),
            pl.BlockSpec((2, _K), lambda i: (0, 0)),
        ],
        out_specs=(
            pl.BlockSpec((_TILE_M2, _D), lambda i: (i, 0)),
            pl.BlockSpec((1, 1), lambda i: (0, 0)),
            pl.BlockSpec((1, 1), lambda i: (0, 0)),
        ),
        out_shape=(
            jax.ShapeDtypeStruct((_N, _D), jnp.float32),
            jax.ShapeDtypeStruct((1, 1), jnp.float32),
            jax.ShapeDtypeStruct((1, 1), jnp.float32),
        ),
        scratch_shapes=[pltpu.SMEM((1,), jnp.float32)],
    )(xf, q, counts2)


def kernel(x, embedding):
    xf = x.reshape(-1, _D)
    x2 = jnp.sum(xf ** 2, axis=1, keepdims=True)
    cn = jnp.sum(embedding ** 2, axis=0, keepdims=True)
    idx3 = _tc_argmin(xf, x2, cn, embedding)
    idx = idx3.reshape(-1)
    table = embedding.astype(jnp.bfloat16).astype(jnp.float32).T
    sc_kernel, nc = _make_sc_gather_hist()
    q, counts2 = sc_kernel(table, idx,
                           jnp.ones((_N,), jnp.float32),
                           jnp.zeros((_K,), jnp.float32))
    qst, loss, perp = _tc_finish(xf, q, counts2)
    return (loss[0, 0], qst.reshape(x.shape), perp[0, 0],
            idx3.reshape(x.shape[:-1]))


# speedup vs baseline: 1.3878x; 1.0237x over previous
import functools

import jax
import jax.numpy as jnp
from jax import lax
from jax.experimental import pallas as pl
from jax.experimental.pallas import tpu as pltpu
from jax.experimental.pallas import tpu_sc as plsc

_D = 32
_K = 8192
_N = 8192
_COMMITMENT = 0.25
_TILE_M = 256
_CHUNK = 2048
_TILE_M2 = 1024


def _argmin_body(x_ref, x2_ref, cn_ref, emb_ref, idx_ref, table_ref):
    x = x_ref[...]
    emb = emb_ref[...]
    s_n2 = lax.dot_general((-2.0 * x), emb, (((1,), (0,)), ((), ())),
                           preferred_element_type=jnp.float32)
    dist = (x2_ref[...] + s_n2) + cn_ref[...]
    iota = lax.broadcasted_iota(jnp.int32, dist.shape, 1)
    best_v = jnp.full((dist.shape[0], 1), -jnp.inf, jnp.float32)
    best_i = jnp.zeros((dist.shape[0], 1), jnp.int32)
    for t in range(_K // _CHUNK):
        sl = dist[:, t * _CHUNK:(t + 1) * _CHUNK]
        io = iota[:, t * _CHUNK:(t + 1) * _CHUNK]
        mn = jnp.min(sl, axis=1, keepdims=True)
        ci = jnp.min(jnp.where(sl == mn, io, _K), axis=1, keepdims=True)
        m = -mn
        take = m > best_v
        best_v = jnp.where(take, m.astype(jnp.bfloat16).astype(jnp.float32),
                           best_v)
        best_i = jnp.where(take, ci, best_i)
    idx_ref[0, 0, :] = best_i[:, 0]
    i = pl.program_id(0)
    esl = emb_ref[:, pl.ds(i * _TILE_M, _TILE_M)]
    table_ref[...] = esl.astype(jnp.bfloat16).astype(jnp.float32).T


def _tc_argmin(xf, x2, cn, embedding):
    grid = _N // _TILE_M
    return pl.pallas_call(
        _argmin_body,
        grid=(grid,),
        in_specs=[
            pl.BlockSpec((_TILE_M, _D), lambda i: (i, 0)),
            pl.BlockSpec((_TILE_M, 1), lambda i: (i, 0)),
            pl.BlockSpec((1, _K), lambda i: (0, 0)),
            pl.BlockSpec((_D, _K), lambda i: (0, 0)),
        ],
        out_specs=(
            pl.BlockSpec((1, 1, _TILE_M), lambda i: (i, 0, 0)),
            pl.BlockSpec((_TILE_M, _D), lambda i: (i, 0)),
        ),
        out_shape=(
            jax.ShapeDtypeStruct((grid, 1, _TILE_M), jnp.int32),
            jax.ShapeDtypeStruct((_K, _D), jnp.float32),
        ),
    )(xf, x2, cn, embedding)


def _make_sc_gather_hist():
    info = plsc.get_sparse_core_info()
    nc, ns = info.num_cores, info.num_subcores
    nw = nc * ns
    bpw = _N // nw
    mesh = plsc.VectorSubcoreMesh(core_axis_name="c", subcore_axis_name="s")

    @functools.partial(
        pl.kernel, mesh=mesh,
        compiler_params=pltpu.CompilerParams(use_tc_tiling_on_sc=False),
        out_type=(jax.ShapeDtypeStruct((_N, _D), jnp.float32),
                  jax.ShapeDtypeStruct((nc, _K), jnp.float32)),
        scratch_types=[
            pltpu.VMEM((bpw,), jnp.int32),
            pltpu.VMEM((bpw, _D), jnp.float32),
            pltpu.VMEM((bpw,), jnp.float32),
            pltpu.VMEM((_K // ns,), jnp.float32),
            pltpu.VMEM_SHARED((_K,), jnp.float32),
            pltpu.SemaphoreType.DMA,
        ],
    )
    def sc_kernel(table_hbm, idx_hbm,
                  q_hbm, counts_hbm, idx_v, rows_v, ones_v, zeros_v,
                  shared, sem):
        cid = lax.axis_index("c")
        sid = lax.axis_index("s")
        wid = sid * nc + cid
        base = wid * bpw
        for j in range(bpw // 16):
            ones_v[pl.ds(j * 16, 16)] = jnp.full((16,), 1.0, jnp.float32)
        for j in range((_K // ns) // 16):
            zeros_v[pl.ds(j * 16, 16)] = jnp.full((16,), 0.0, jnp.float32)
        pltpu.sync_copy(zeros_v, shared.at[pl.ds(sid * (_K // ns), _K // ns)])
        pltpu.sync_copy(idx_hbm.at[pl.ds(base, bpw)], idx_v)
        pltpu.async_copy(table_hbm.at[idx_v], rows_v, sem).wait()
        pltpu.sync_copy(rows_v, q_hbm.at[pl.ds(base, bpw)])
        plsc.subcore_barrier()
        pltpu.sync_copy(ones_v, shared.at[idx_v], add=True)
        plsc.subcore_barrier()
        @pl.when(sid == 0)
        def _():
            pltpu.sync_copy(shared, counts_hbm.at[cid])

    return sc_kernel, nc


def _finish_body(x_ref, q_ref, counts_ref, qst_ref, loss_ref, perp_ref,
                 sse_ref):
    i = pl.program_id(0)
    n = pl.num_programs(0)
    x = x_ref[...]
    q = q_ref[...]
    qst_ref[...] = x + (q - x)
    diff = q - x
    part = jnp.sum(diff * diff)

    @pl.when(i == 0)
    def _():
        sse_ref[0] = part

    @pl.when(i > 0)
    def _():
        sse_ref[0] = sse_ref[0] + part

    @pl.when(i == n - 1)
    def _():
        m = sse_ref[0] / (_N * _D)
        loss_ref[...] = jnp.broadcast_to(m + _COMMITMENT * m, (1, 1))
        counts = counts_ref[0:1, :] + counts_ref[1:2, :]
        p = counts * (1.0 / _N)
        ent = jnp.sum(p * jnp.log(p + 1e-10), axis=1, keepdims=True)
        perp_ref[...] = jnp.exp(-ent)


def _tc_finish(xf, q, counts2):
    grid = _N // _TILE_M2
    return pl.pallas_call(
        _finish_body,
        grid=(grid,),
        in_specs=[
            pl.BlockSpec((_TILE_M2, _D), lambda i: (i, 0)),
            pl.BlockSpec((_TILE_M2, _D), lambda i: (i, 0)),
            pl.BlockSpec((2, _K), lambda i: (0, 0)),
        ],
        out_specs=(
            pl.BlockSpec((_TILE_M2, _D), lambda i: (i, 0)),
            pl.BlockSpec((1, 1), lambda i: (0, 0)),
            pl.BlockSpec((1, 1), lambda i: (0, 0)),
        ),
        out_shape=(
            jax.ShapeDtypeStruct((_N, _D), jnp.float32),
            jax.ShapeDtypeStruct((1, 1), jnp.float32),
            jax.ShapeDtypeStruct((1, 1), jnp.float32),
        ),
        scratch_shapes=[pltpu.SMEM((1,), jnp.float32)],
    )(xf, q, counts2)


def kernel(x, embedding):
    xf = x.reshape(-1, _D)
    x2 = jnp.sum(xf ** 2, axis=1, keepdims=True)
    cn = jnp.sum(embedding ** 2, axis=0, keepdims=True)
    idx3, table = _tc_argmin(xf, x2, cn, embedding)
    idx = idx3.reshape(-1)
    sc_kernel, nc = _make_sc_gather_hist()
    q, counts2 = sc_kernel(table, idx)
    qst, loss, perp = _tc_finish(xf, q, counts2)
    return (loss[0, 0], qst.reshape(x.shape), perp[0, 0],
            idx3.reshape(x.shape[:-1]))


# speedup vs baseline: 1.4662x; 1.0565x over previous
import functools

import jax
import jax.numpy as jnp
from jax import lax
from jax.experimental import pallas as pl
from jax.experimental.pallas import tpu as pltpu
from jax.experimental.pallas import tpu_sc as plsc

_D = 32
_K = 8192
_N = 8192
_COMMITMENT = 0.25
_TILE_M = 256
_CHUNK = 2048
_TILE_M2 = 1024


def _argmin_body(x_ref, x2_ref, cn_ref, emb_ref, idx_ref, table_ref):
    x = x_ref[...]
    emb = emb_ref[...]
    s_n2 = lax.dot_general((-2.0 * x), emb, (((1,), (0,)), ((), ())),
                           preferred_element_type=jnp.float32)
    dist = (x2_ref[...] + s_n2) + cn_ref[...]
    best_v = jnp.full((dist.shape[0], 1), -jnp.inf, jnp.float32)
    best_i = jnp.zeros((dist.shape[0], 1), jnp.int32)
    for t in range(_K // _CHUNK):
        sl = dist[:, t * _CHUNK:(t + 1) * _CHUNK]
        mn = jnp.min(sl, axis=1, keepdims=True)
        ci = (jnp.argmin(sl, axis=1).astype(jnp.int32)
              + t * _CHUNK)[:, None]
        m = -mn
        take = m > best_v
        best_v = jnp.where(take, m.astype(jnp.bfloat16).astype(jnp.float32),
                           best_v)
        best_i = jnp.where(take, ci, best_i)
    idx_ref[0, 0, :] = best_i[:, 0]
    i = pl.program_id(0)
    esl = emb_ref[:, pl.ds(i * _TILE_M, _TILE_M)]
    table_ref[...] = esl.astype(jnp.bfloat16).astype(jnp.float32).T


def _tc_argmin(xf, x2, cn, embedding):
    grid = _N // _TILE_M
    return pl.pallas_call(
        _argmin_body,
        grid=(grid,),
        in_specs=[
            pl.BlockSpec((_TILE_M, _D), lambda i: (i, 0)),
            pl.BlockSpec((_TILE_M, 1), lambda i: (i, 0)),
            pl.BlockSpec((1, _K), lambda i: (0, 0)),
            pl.BlockSpec((_D, _K), lambda i: (0, 0)),
        ],
        out_specs=(
            pl.BlockSpec((1, 1, _TILE_M), lambda i: (i, 0, 0)),
            pl.BlockSpec((_TILE_M, _D), lambda i: (i, 0)),
        ),
        out_shape=(
            jax.ShapeDtypeStruct((grid, 1, _TILE_M), jnp.int32),
            jax.ShapeDtypeStruct((_K, _D), jnp.float32),
        ),
    )(xf, x2, cn, embedding)


def _make_sc_gather_hist():
    info = plsc.get_sparse_core_info()
    nc, ns = info.num_cores, info.num_subcores
    nw = nc * ns
    bpw = _N // nw
    mesh = plsc.VectorSubcoreMesh(core_axis_name="c", subcore_axis_name="s")

    @functools.partial(
        pl.kernel, mesh=mesh,
        compiler_params=pltpu.CompilerParams(use_tc_tiling_on_sc=False),
        out_type=(jax.ShapeDtypeStruct((_N, _D), jnp.float32),
                  jax.ShapeDtypeStruct((nc, _K), jnp.float32)),
        scratch_types=[
            pltpu.VMEM((bpw,), jnp.int32),
            pltpu.VMEM((bpw, _D), jnp.float32),
            pltpu.VMEM((bpw,), jnp.float32),
            pltpu.VMEM((_K // ns,), jnp.float32),
            pltpu.VMEM_SHARED((_K,), jnp.float32),
            pltpu.SemaphoreType.DMA,
        ],
    )
    def sc_kernel(table_hbm, idx_hbm,
                  q_hbm, counts_hbm, idx_v, rows_v, ones_v, zeros_v,
                  shared, sem):
        cid = lax.axis_index("c")
        sid = lax.axis_index("s")
        wid = sid * nc + cid
        base = wid * bpw
        for j in range(bpw // 16):
            ones_v[pl.ds(j * 16, 16)] = jnp.full((16,), 1.0, jnp.float32)
        for j in range((_K // ns) // 16):
            zeros_v[pl.ds(j * 16, 16)] = jnp.full((16,), 0.0, jnp.float32)
        pltpu.sync_copy(zeros_v, shared.at[pl.ds(sid * (_K // ns), _K // ns)])
        pltpu.sync_copy(idx_hbm.at[pl.ds(base, bpw)], idx_v)
        pltpu.async_copy(table_hbm.at[idx_v], rows_v, sem).wait()
        pltpu.sync_copy(rows_v, q_hbm.at[pl.ds(base, bpw)])
        plsc.subcore_barrier()
        pltpu.sync_copy(ones_v, shared.at[idx_v], add=True)
        plsc.subcore_barrier()
        @pl.when(sid == 0)
        def _():
            pltpu.sync_copy(shared, counts_hbm.at[cid])

    return sc_kernel, nc


def _finish_body(x_ref, q_ref, counts_ref, qst_ref, loss_ref, perp_ref,
                 sse_ref):
    i = pl.program_id(0)
    n = pl.num_programs(0)
    x = x_ref[...]
    q = q_ref[...]
    qst_ref[...] = x + (q - x)
    diff = q - x
    part = jnp.sum(diff * diff)

    @pl.when(i == 0)
    def _():
        sse_ref[0] = part

    @pl.when(i > 0)
    def _():
        sse_ref[0] = sse_ref[0] + part

    @pl.when(i == n - 1)
    def _():
        m = sse_ref[0] / (_N * _D)
        loss_ref[...] = jnp.broadcast_to(m + _COMMITMENT * m, (1, 1))
        counts = counts_ref[0:1, :] + counts_ref[1:2, :]
        p = counts * (1.0 / _N)
        ent = jnp.sum(p * jnp.log(p + 1e-10), axis=1, keepdims=True)
        perp_ref[...] = jnp.exp(-ent)


def _tc_finish(xf, q, counts2):
    grid = _N // _TILE_M2
    return pl.pallas_call(
        _finish_body,
        grid=(grid,),
        in_specs=[
            pl.BlockSpec((_TILE_M2, _D), lambda i: (i, 0)),
            pl.BlockSpec((_TILE_M2, _D), lambda i: (i, 0)),
            pl.BlockSpec((2, _K), lambda i: (0, 0)),
        ],
        out_specs=(
            pl.BlockSpec((_TILE_M2, _D), lambda i: (i, 0)),
            pl.BlockSpec((1, 1), lambda i: (0, 0)),
            pl.BlockSpec((1, 1), lambda i: (0, 0)),
        ),
        out_shape=(
            jax.ShapeDtypeStruct((_N, _D), jnp.float32),
            jax.ShapeDtypeStruct((1, 1), jnp.float32),
            jax.ShapeDtypeStruct((1, 1), jnp.float32),
        ),
        scratch_shapes=[pltpu.SMEM((1,), jnp.float32)],
    )(xf, q, counts2)


def kernel(x, embedding):
    xf = x.reshape(-1, _D)
    x2 = jnp.sum(xf ** 2, axis=1, keepdims=True)
    cn = jnp.sum(embedding ** 2, axis=0, keepdims=True)
    idx3, table = _tc_argmin(xf, x2, cn, embedding)
    idx = idx3.reshape(-1)
    sc_kernel, nc = _make_sc_gather_hist()
    q, counts2 = sc_kernel(table, idx)
    qst, loss, perp = _tc_finish(xf, q, counts2)
    return (loss[0, 0], qst.reshape(x.shape), perp[0, 0],
            idx3.reshape(x.shape[:-1]))


# speedup vs baseline: 1.5641x; 1.0668x over previous
import functools

import jax
import jax.numpy as jnp
from jax import lax
from jax.experimental import pallas as pl
from jax.experimental.pallas import tpu as pltpu
from jax.experimental.pallas import tpu_sc as plsc

_D = 32
_K = 8192
_N = 8192
_COMMITMENT = 0.25
_TILE_M = 256
_CHUNK = 2048
_DPAD = 128
_TILE_M2 = 1024


def _argmin_body(x_ref, x2_ref, cn_ref, emb_ref, idx_ref, table_ref):
    x = x_ref[...]
    emb = emb_ref[...]
    s_n2 = lax.dot_general((-2.0 * x), emb, (((1,), (0,)), ((), ())),
                           preferred_element_type=jnp.float32)
    dist = (x2_ref[...] + s_n2) + cn_ref[...]
    best_v = jnp.full((dist.shape[0], 1), -jnp.inf, jnp.float32)
    best_i = jnp.zeros((dist.shape[0], 1), jnp.int32)
    for t in range(_K // _CHUNK):
        sl = dist[:, t * _CHUNK:(t + 1) * _CHUNK]
        mn = jnp.min(sl, axis=1, keepdims=True)
        ci = (jnp.argmin(sl, axis=1).astype(jnp.int32)
              + t * _CHUNK)[:, None]
        m = -mn
        take = m > best_v
        best_v = jnp.where(take, m.astype(jnp.bfloat16).astype(jnp.float32),
                           best_v)
        best_i = jnp.where(take, ci, best_i)
    idx_ref[0, 0, :] = best_i[:, 0]
    i = pl.program_id(0)
    esl = emb_ref[:, pl.ds(i * _TILE_M, _TILE_M)]
    tb = esl.astype(jnp.bfloat16).astype(jnp.float32).T
    table_ref[...] = jnp.concatenate(
        [tb, jnp.zeros((tb.shape[0], _DPAD - _D), jnp.float32)], axis=1)


def _tc_argmin(xf, x2, cn, embedding):
    grid = _N // _TILE_M
    return pl.pallas_call(
        _argmin_body,
        grid=(grid,),
        in_specs=[
            pl.BlockSpec((_TILE_M, _D), lambda i: (i, 0)),
            pl.BlockSpec((_TILE_M, 1), lambda i: (i, 0)),
            pl.BlockSpec((1, _K), lambda i: (0, 0)),
            pl.BlockSpec((_D, _K), lambda i: (0, 0)),
        ],
        out_specs=(
            pl.BlockSpec((1, 1, _TILE_M), lambda i: (i, 0, 0)),
            pl.BlockSpec((_TILE_M, _DPAD), lambda i: (i, 0)),
        ),
        out_shape=(
            jax.ShapeDtypeStruct((grid, 1, _TILE_M), jnp.int32),
            jax.ShapeDtypeStruct((_K, _DPAD), jnp.float32),
        ),
    )(xf, x2, cn, embedding)


def _make_sc_gather_hist():
    info = plsc.get_sparse_core_info()
    nc, ns = info.num_cores, info.num_subcores
    nw = nc * ns
    bpw = _N // nw
    mesh = plsc.VectorSubcoreMesh(core_axis_name="c", subcore_axis_name="s")

    @functools.partial(
        pl.kernel, mesh=mesh,
        out_type=(jax.ShapeDtypeStruct((_N, _DPAD), jnp.float32),
                  jax.ShapeDtypeStruct((nc * _K,), jnp.float32)),
        scratch_types=[
            pltpu.VMEM((bpw,), jnp.int32),
            pltpu.VMEM((bpw, _DPAD), jnp.float32),
            pltpu.VMEM((bpw,), jnp.float32),
            pltpu.VMEM((_K // ns,), jnp.float32),
            pltpu.VMEM_SHARED((_K,), jnp.float32),
            pltpu.SemaphoreType.DMA,
        ],
    )
    def sc_kernel(table_hbm, idx_hbm,
                  q_hbm, counts_hbm, idx_v, rows_v, ones_v, zeros_v,
                  shared, sem):
        cid = lax.axis_index("c")
        sid = lax.axis_index("s")
        wid = sid * nc + cid
        base = wid * bpw
        for j in range(bpw // 16):
            ones_v[pl.ds(j * 16, 16)] = jnp.full((16,), 1.0, jnp.float32)
        for j in range((_K // ns) // 16):
            zeros_v[pl.ds(j * 16, 16)] = jnp.full((16,), 0.0, jnp.float32)
        pltpu.sync_copy(zeros_v, shared.at[pl.ds(sid * (_K // ns), _K // ns)])
        pltpu.sync_copy(idx_hbm.at[pl.ds(base, bpw)], idx_v)
        pltpu.async_copy(table_hbm.at[idx_v], rows_v, sem).wait()
        pltpu.sync_copy(rows_v, q_hbm.at[pl.ds(base, bpw)])
        plsc.subcore_barrier()
        pltpu.sync_copy(ones_v, shared.at[idx_v], add=True)
        plsc.subcore_barrier()
        @pl.when(sid == 0)
        def _():
            pltpu.sync_copy(shared, counts_hbm.at[pl.ds(cid * _K, _K)])

    return sc_kernel, nc


def _finish_body(x_ref, q_ref, counts_ref, qst_ref, loss_ref, perp_ref,
                 sse_ref):
    i = pl.program_id(0)
    n = pl.num_programs(0)
    x = x_ref[...]
    q = q_ref[:, 0:_D]
    qst_ref[...] = x + (q - x)
    diff = q - x
    part = jnp.sum(diff * diff)

    @pl.when(i == 0)
    def _():
        sse_ref[0] = part

    @pl.when(i > 0)
    def _():
        sse_ref[0] = sse_ref[0] + part

    @pl.when(i == n - 1)
    def _():
        m = sse_ref[0] / (_N * _D)
        loss_ref[...] = jnp.broadcast_to(m + _COMMITMENT * m, (1, 1))
        counts = counts_ref[:, 0:_K] + counts_ref[:, _K:2 * _K]
        p = counts * (1.0 / _N)
        ent = jnp.sum(p * jnp.log(p + 1e-10), axis=1, keepdims=True)
        perp_ref[...] = jnp.exp(-ent)


def _tc_finish(xf, q, counts2):
    grid = _N // _TILE_M2
    return pl.pallas_call(
        _finish_body,
        grid=(grid,),
        in_specs=[
            pl.BlockSpec((_TILE_M2, _D), lambda i: (i, 0)),
            pl.BlockSpec((_TILE_M2, _DPAD), lambda i: (i, 0)),
            pl.BlockSpec((1, 2 * _K), lambda i: (0, 0)),
        ],
        out_specs=(
            pl.BlockSpec((_TILE_M2, _D), lambda i: (i, 0)),
            pl.BlockSpec((1, 1), lambda i: (0, 0)),
            pl.BlockSpec((1, 1), lambda i: (0, 0)),
        ),
        out_shape=(
            jax.ShapeDtypeStruct((_N, _D), jnp.float32),
            jax.ShapeDtypeStruct((1, 1), jnp.float32),
            jax.ShapeDtypeStruct((1, 1), jnp.float32),
        ),
        scratch_shapes=[pltpu.SMEM((1,), jnp.float32)],
    )(xf, q, counts2)


def kernel(x, embedding):
    xf = x.reshape(-1, _D)
    x2 = jnp.sum(xf ** 2, axis=1, keepdims=True)
    cn = jnp.sum(embedding ** 2, axis=0, keepdims=True)
    idx3, table = _tc_argmin(xf, x2, cn, embedding)
    idx = idx3.reshape(-1)
    sc_kernel, nc = _make_sc_gather_hist()
    q, counts2 = sc_kernel(table, idx)
    qst, loss, perp = _tc_finish(xf, q, counts2.reshape(1, nc * _K))
    return (loss[0, 0], qst.reshape(x.shape), perp[0, 0],
            idx3.reshape(x.shape[:-1]))


# speedup vs baseline: 1.5642x; 1.0000x over previous
import functools

import jax
import jax.numpy as jnp
from jax import lax
from jax.experimental import pallas as pl
from jax.experimental.pallas import tpu as pltpu
from jax.experimental.pallas import tpu_sc as plsc

_D = 32
_K = 8192
_N = 8192
_COMMITMENT = 0.25
_TILE_M = 256
_CHUNK = 2048
_DPAD = 128
_TILE_M2 = 1024


def _argmin_body(x_ref, x2_ref, cn_ref, emb_ref, idx_ref, table_ref):
    x = x_ref[...]
    emb = emb_ref[...]
    s_n2 = lax.dot_general((-2.0 * x), emb, (((1,), (0,)), ((), ())),
                           preferred_element_type=jnp.float32)
    dist = (x2_ref[...] + s_n2) + cn_ref[...]
    best_v = jnp.full((dist.shape[0], 1), -jnp.inf, jnp.float32)
    best_i = jnp.zeros((dist.shape[0], 1), jnp.int32)
    for t in range(_K // _CHUNK):
        sl = dist[:, t * _CHUNK:(t + 1) * _CHUNK]
        mn = jnp.min(sl, axis=1, keepdims=True)
        ci = (jnp.argmin(sl, axis=1).astype(jnp.int32)
              + t * _CHUNK)[:, None]
        m = -mn
        take = m > best_v
        best_v = jnp.where(take, m.astype(jnp.bfloat16).astype(jnp.float32),
                           best_v)
        best_i = jnp.where(take, ci, best_i)
    idx_ref[...] = best_i[:, 0]
    i = pl.program_id(0)
    esl = emb_ref[:, pl.ds(i * _TILE_M, _TILE_M)]
    tb = esl.astype(jnp.bfloat16).astype(jnp.float32).T
    table_ref[...] = jnp.concatenate(
        [tb, jnp.zeros((tb.shape[0], _DPAD - _D), jnp.float32)], axis=1)


def _tc_argmin(xf, x2, cn, embedding):
    grid = _N // _TILE_M
    return pl.pallas_call(
        _argmin_body,
        grid=(grid,),
        in_specs=[
            pl.BlockSpec((_TILE_M, _D), lambda i: (i, 0)),
            pl.BlockSpec((_TILE_M, 1), lambda i: (i, 0)),
            pl.BlockSpec((1, _K), lambda i: (0, 0)),
            pl.BlockSpec((_D, _K), lambda i: (0, 0)),
        ],
        out_specs=(
            pl.BlockSpec((_TILE_M,), lambda i: (i,)),
            pl.BlockSpec((_TILE_M, _DPAD), lambda i: (i, 0)),
        ),
        out_shape=(
            jax.ShapeDtypeStruct((_N,), jnp.int32),
            jax.ShapeDtypeStruct((_K, _DPAD), jnp.float32),
        ),
    )(xf, x2, cn, embedding)


def _make_sc_gather_hist():
    info = plsc.get_sparse_core_info()
    nc, ns = info.num_cores, info.num_subcores
    nw = nc * ns
    bpw = _N // nw
    mesh = plsc.VectorSubcoreMesh(core_axis_name="c", subcore_axis_name="s")

    @functools.partial(
        pl.kernel, mesh=mesh,
        out_type=(jax.ShapeDtypeStruct((_N, _DPAD), jnp.float32),
                  jax.ShapeDtypeStruct((nc * _K,), jnp.float32)),
        scratch_types=[
            pltpu.VMEM((bpw,), jnp.int32),
            pltpu.VMEM((bpw, _DPAD), jnp.float32),
            pltpu.VMEM((bpw,), jnp.float32),
            pltpu.VMEM((_K // ns,), jnp.float32),
            pltpu.VMEM_SHARED((_K,), jnp.float32),
            pltpu.SemaphoreType.DMA,
        ],
    )
    def sc_kernel(table_hbm, idx_hbm,
                  q_hbm, counts_hbm, idx_v, rows_v, ones_v, zeros_v,
                  shared, sem):
        cid = lax.axis_index("c")
        sid = lax.axis_index("s")
        wid = sid * nc + cid
        base = wid * bpw
        for j in range(bpw // 16):
            ones_v[pl.ds(j * 16, 16)] = jnp.full((16,), 1.0, jnp.float32)
        for j in range((_K // ns) // 16):
            zeros_v[pl.ds(j * 16, 16)] = jnp.full((16,), 0.0, jnp.float32)
        pltpu.sync_copy(zeros_v, shared.at[pl.ds(sid * (_K // ns), _K // ns)])
        pltpu.sync_copy(idx_hbm.at[pl.ds(base, bpw)], idx_v)
        pltpu.async_copy(table_hbm.at[idx_v], rows_v, sem).wait()
        pltpu.sync_copy(rows_v, q_hbm.at[pl.ds(base, bpw)])
        plsc.subcore_barrier()
        pltpu.sync_copy(ones_v, shared.at[idx_v], add=True)
        plsc.subcore_barrier()
        @pl.when(sid == 0)
        def _():
            pltpu.sync_copy(shared, counts_hbm.at[pl.ds(cid * _K, _K)])

    return sc_kernel, nc


def _finish_body(x_ref, q_ref, counts_ref, qst_ref, loss_ref, perp_ref,
                 sse_ref):
    i = pl.program_id(0)
    n = pl.num_programs(0)
    x = x_ref[...]
    q = q_ref[:, 0:_D]
    qst_ref[...] = x + (q - x)
    diff = q - x
    part = jnp.sum(diff * diff)

    @pl.when(i == 0)
    def _():
        sse_ref[0] = part

    @pl.when(i > 0)
    def _():
        sse_ref[0] = sse_ref[0] + part

    @pl.when(i == n - 1)
    def _():
        m = sse_ref[0] / (_N * _D)
        loss_ref[...] = jnp.broadcast_to(m + _COMMITMENT * m, (1, 1))
        counts = counts_ref[:, 0:_K] + counts_ref[:, _K:2 * _K]
        p = counts * (1.0 / _N)
        ent = jnp.sum(p * jnp.log(p + 1e-10), axis=1, keepdims=True)
        perp_ref[...] = jnp.exp(-ent)


def _tc_finish(xf, q, counts2):
    grid = _N // _TILE_M2
    return pl.pallas_call(
        _finish_body,
        grid=(grid,),
        in_specs=[
            pl.BlockSpec((_TILE_M2, _D), lambda i: (i, 0)),
            pl.BlockSpec((_TILE_M2, _DPAD), lambda i: (i, 0)),
            pl.BlockSpec((1, 2 * _K), lambda i: (0, 0)),
        ],
        out_specs=(
            pl.BlockSpec((_TILE_M2, _D), lambda i: (i, 0)),
            pl.BlockSpec((1, 1), lambda i: (0, 0)),
            pl.BlockSpec((1, 1), lambda i: (0, 0)),
        ),
        out_shape=(
            jax.ShapeDtypeStruct((_N, _D), jnp.float32),
            jax.ShapeDtypeStruct((1, 1), jnp.float32),
            jax.ShapeDtypeStruct((1, 1), jnp.float32),
        ),
        scratch_shapes=[pltpu.SMEM((1,), jnp.float32)],
    )(xf, q, counts2)


def kernel(x, embedding):
    xf = x.reshape(-1, _D)
    x2 = jnp.sum(xf ** 2, axis=1, keepdims=True)
    cn = jnp.sum(embedding ** 2, axis=0, keepdims=True)
    idx, table = _tc_argmin(xf, x2, cn, embedding)
    sc_kernel, nc = _make_sc_gather_hist()
    q, counts2 = sc_kernel(table, idx)
    qst, loss, perp = _tc_finish(xf, q, counts2.reshape(1, nc * _K))
    return (loss[0, 0], qst.reshape(x.shape), perp[0, 0],
            idx.reshape(x.shape[:-1]))
